# baseline (speedup 1.0000x reference)
import functools

import jax
import jax.numpy as jnp
from jax import lax
from jax.experimental import pallas as pl
from jax.experimental.pallas import tpu as pltpu
from jax.experimental.pallas import tpu_sc as plsc

NUM_EMB = 1000000
DIM = 64
BATCH = 16384

_TILE_COLS = 128 * ((NUM_EMB + 127) // 128)
_COLS_PER_W = _TILE_COLS // 32


@functools.cache
def _build():
    info = plsc.get_sparse_core_info()
    NC, NS = info.num_cores, info.num_subcores
    NW = NC * NS
    bpw = BATCH // NW

    CPW = 31360
    PIECE = 128
    NP = CPW // PIECE
    NSUP = 31
    DUMP = BATCH
    OUT_ROWS = BATCH + 32

    mesh = plsc.VectorSubcoreMesh(core_axis_name="c", subcore_axis_name="s")

    @functools.partial(
        pl.kernel,
        mesh=mesh,
        compiler_params=pltpu.CompilerParams(needs_layout_passes=False),
        out_type=jax.ShapeDtypeStruct((OUT_ROWS, 128), jnp.float32),
        scratch_types=[
            pltpu.VMEM((4096,), jnp.int32),
            pltpu.VMEM((BATCH + 16,), jnp.int32),
            pltpu.VMEM((BATCH + 16,), jnp.int32),
            pltpu.VMEM((48,), jnp.int32),
            pltpu.VMEM((4, DIM, PIECE), jnp.float32),
            pltpu.VMEM((DIM, 128), jnp.float32),
            pltpu.VMEM((4, 16), jnp.int32),
            pltpu.SMEM((40,), jnp.int32),
            pltpu.SemaphoreType.DMA,
            pltpu.SemaphoreType.DMA,
        ],
    )
    def gather_stream(
        tt, idx_hbm, out, idxc, l1, l2, cntv, dbuf, rowbuf, sidx, smso,
        sem_f, sem_s,
    ):
        wid = lax.axis_index("s") * NC + lax.axis_index("c")
        lo = wid * CPW
        lanes = lax.iota(jnp.int32, 16)
        is_last = wid == NW - 1
        npieces = jnp.where(is_last, 218, NP)

        off = 0
        for big in range(BATCH // 4096):
            pltpu.sync_copy(idx_hbm.at[pl.ds(big * 4096, 4096)], idxc)

            def scan(k, off, big=big):
                c = idxc[pl.ds(k * 16, 16)]
                crel = c - lo
                m = (crel >= 0) & (crel < CPW)
                b16 = big * 4096 + k * 16 + lanes
                packed = (crel << 14) | b16
                plsc.store_compressed(l1.at[pl.ds(off, 16)], packed, mask=m)
                pc = plsc.all_reduce_population_count(m)
                return off + pc[0]

            off = lax.fori_loop(0, 4096 // 16, scan, off)
        cnt = off

        nch = (cnt + 15) >> 4
        zero16 = jnp.zeros((16,), jnp.int32)
        for z in range(3):
            cntv[pl.ds(z * 16, 16)] = zero16

        def count(t, carry):
            e = l1[pl.ds(t * 16, 16)]
            valid = (t * 16 + lanes) < cnt
            s = e >> 24
            for i in range(NSUP):
                mi = valid & (s == i)
                pc = plsc.all_reduce_population_count(mi)
                plsc.addupdate_scatter(
                    cntv, [jnp.full((16,), i, jnp.int32)], pc, mask=lanes == 0
                )
            return carry

        lax.fori_loop(0, nch, count, 0)

        cvec = [cntv[pl.ds(z * 16, 16)] for z in range(2)]
        so = [0] * (NSUP + 1)
        for i in range(NSUP):
            so[i + 1] = so[i] + cvec[i // 16][i % 16]
        for i in range(NSUP + 1):
            smso[i] = so[i]

        def redist(t, woffs):
            e = l1[pl.ds(t * 16, 16)]
            valid = (t * 16 + lanes) < cnt
            s = e >> 24
            new = []
            for i in range(NSUP):
                mi = valid & (s == i)
                plsc.store_compressed(l2.at[pl.ds(woffs[i], 16)], e, mask=mi)
                pc = plsc.all_reduce_population_count(mi)
                new.append(woffs[i] + pc[0])
            return tuple(new)

        lax.fori_loop(0, nch, redist, tuple(so[:NSUP]))

        def fetch_piece(q, r):
            @pl.when(q < npieces)
            def _f():
                pltpu.async_copy(
                    tt.at[:, pl.ds(lo + q * PIECE, PIECE)], dbuf.at[r], sem_f
                )

        for r in range(4):
            fetch_piece(r, r)

        def piece_body(p, r, ring):
            buf = dbuf.at[r]

            @pl.when(p < npieces)
            def _w():
                pltpu.make_async_copy(
                    tt.at[:, pl.ds(0, PIECE)], buf, sem_f
                ).wait()

            width = jnp.where(p < npieces, PIECE, 0)
            pbase = p * PIECE
            sup = p >> 3
            sbeg = smso[sup]
            send = smso[sup + 1]
            nseg = (send - sbeg + 15) >> 4

            def visit(u, ring):
                eo = sbeg + u * 16
                e = l2[pl.ds(eo, 16)]
                valid = (eo + lanes) < send
                crel = e >> 14
                cc = crel - pbase
                m = valid & (cc >= 0) & (cc < width)
                slot = ring & 3

                @pl.when(ring >= 4)
                def _ws():
                    pltpu.make_async_copy(
                        rowbuf.at[pl.ds(0, 16), :],
                        out.at[sidx.at[0]], sem_s
                    ).wait()

                ccs = jnp.clip(cc, 0, PIECE - 1)
                bv = jnp.where(m, e & 16383, DUMP + ((lanes + wid) & 31))
                plsc.store_scatter(
                    sidx, [jnp.full((16,), slot, jnp.int32), lanes], bv
                )
                rrows = slot * 16 + lanes
                for j in range(DIM):
                    x = plsc.load_gather(
                        buf, [jnp.full((16,), j, jnp.int32), ccs]
                    )
                    plsc.store_scatter(
                        rowbuf, [rrows, jnp.full((16,), j, jnp.int32)], x
                    )
                pltpu.async_copy(
                    rowbuf.at[pl.ds(slot * 16, 16), :],
                    out.at[sidx.at[slot]], sem_s
                )
                return ring + 1

            ring = lax.fori_loop(0, nseg, visit, ring)
            fetch_piece(p + 4, r)
            return ring

        def piece_quad(p4, ring):
            for r in range(4):
                ring = piece_body(4 * p4 + r, r, ring)
            return ring

        ring = lax.fori_loop(0, (NP + 3 + 3) // 4, piece_quad, 0)

        def drain(d, carry):
            pltpu.make_async_copy(
                rowbuf.at[pl.ds(0, 16), :], out.at[sidx.at[0]], sem_s
            ).wait()
            return carry

        lax.fori_loop(0, jnp.minimum(ring, 4), drain, 0)

    return gather_stream


def kernel(table, batch_idx):
    f = _build()
    res = f(table.T, batch_idx.astype(jnp.int32))
    return res[:BATCH, :DIM]


# speedup vs baseline: 2.6644x; 2.6644x over previous
import functools

import jax
import jax.numpy as jnp
from jax import lax
from jax.experimental import pallas as pl
from jax.experimental.pallas import tpu as pltpu
from jax.experimental.pallas import tpu_sc as plsc

NUM_EMB = 1000000
DIM = 64
BATCH = 16384


@functools.cache
def _build():
    info = plsc.get_sparse_core_info()
    NC, NS = info.num_cores, info.num_subcores
    NW = NC * NS
    bpw = BATCH // NW
    NBUF = 8

    mesh = plsc.VectorSubcoreMesh(core_axis_name="c", subcore_axis_name="s")

    @functools.partial(
        pl.kernel,
        mesh=mesh,
        compiler_params=pltpu.CompilerParams(needs_layout_passes=False),
        out_type=jax.ShapeDtypeStruct((BATCH, DIM), jnp.float32),
        scratch_types=[
            pltpu.VMEM((bpw + 16,), jnp.int32),
            pltpu.VMEM((NBUF, DIM, 128), jnp.float32),
            pltpu.VMEM((bpw // 2, DIM), jnp.float32),
            pltpu.SemaphoreType.DMA,
        ],
    )
    def gather_cols(tt, idx_hbm, out, idx_v, rbuf, out_v, sem):
        wid = lax.axis_index("s") * NC + lax.axis_index("c")
        base = wid * bpw
        pltpu.sync_copy(idx_hbm.at[pl.ds(base, bpw)], idx_v.at[pl.ds(0, bpw)])

        lanes = lax.iota(jnp.int32, 16)

        def fetch(k, slot):
            tc = idx_v[pl.ds(k, 16)][0] >> 7
            pltpu.async_copy(
                tt.at[:, pl.ds(tc * 128, 128)], rbuf.at[slot], sem
            )

        for k in range(NBUF):
            fetch(k, k)

        def body(k8, carry):
            for r in range(NBUF):
                k = k8 * NBUF + r
                pltpu.make_async_copy(
                    tt.at[:, pl.ds(0, 128)], rbuf.at[r], sem
                ).wait()
                cc = jnp.full((16,), idx_v[pl.ds(k, 16)][0] & 127, jnp.int32)
                kk = jnp.full((16,), k & (bpw // 2 - 1), jnp.int32)
                for g in range(DIM // 16):
                    rows = g * 16 + lanes
                    x = plsc.load_gather(rbuf.at[r], [rows, cc])
                    plsc.store_scatter(out_v, [kk, rows], x)

                @pl.when(k < bpw - NBUF)
                def _next(k=k, r=r):
                    fetch(k + NBUF, r)

            return carry

        half = bpw // 2
        lax.fori_loop(0, half // NBUF, body, 0)
        pltpu.sync_copy(out_v, out.at[pl.ds(base, half), :])
        lax.fori_loop(half // NBUF, bpw // NBUF, body, 0)
        pltpu.sync_copy(out_v, out.at[pl.ds(base + half, half), :])

    return gather_cols


def kernel(table, batch_idx):
    f = _build()
    return f(table.T, batch_idx.astype(jnp.int32))


# speedup vs baseline: 2.6985x; 1.0128x over previous
import functools

import jax
import jax.numpy as jnp
from jax import lax
from jax.experimental import pallas as pl
from jax.experimental.pallas import tpu as pltpu
from jax.experimental.pallas import tpu_sc as plsc

NUM_EMB = 1000000
DIM = 64
BATCH = 16384


@functools.cache
def _build():
    info = plsc.get_sparse_core_info()
    NC, NS = info.num_cores, info.num_subcores
    NW = NC * NS
    bpw = BATCH // NW
    NBUF = 8

    mesh = plsc.VectorSubcoreMesh(core_axis_name="c", subcore_axis_name="s")

    @functools.partial(
        pl.kernel,
        mesh=mesh,
        compiler_params=pltpu.CompilerParams(needs_layout_passes=False),
        out_type=jax.ShapeDtypeStruct((BATCH, DIM), jnp.float32),
        scratch_types=[
            pltpu.VMEM((bpw + 16,), jnp.int32),
            pltpu.VMEM((NBUF, DIM, 128), jnp.float32),
            pltpu.VMEM((bpw // 2, DIM), jnp.float32),
            pltpu.SemaphoreType.DMA,
        ],
    )
    def gather_cols(tt, idx_hbm, out, idx_v, rbuf, out_v, sem):
        wid = lax.axis_index("s") * NC + lax.axis_index("c")
        base = wid * bpw
        pltpu.sync_copy(idx_hbm.at[pl.ds(base, bpw)], idx_v.at[pl.ds(0, bpw)])

        lanes = lax.iota(jnp.int32, 16)

        def fetch(k, slot):
            tc = idx_v[pl.ds(k, 16)][0] >> 7
            pltpu.async_copy(
                tt.at[:, pl.ds(tc * 128, 128)], rbuf.at[slot], sem
            )

        for k in range(NBUF):
            fetch(k, k)

        def body(k8, carry):
            for g4 in range(2):
                pltpu.make_async_copy(
                    tt.at[:, pl.ds(0, 4 * 128)],
                    rbuf.at[pl.ds(g4 * 4, 4)], sem
                ).wait()
                for r4 in range(4):
                    r = g4 * 4 + r4
                    k = k8 * NBUF + r
                    cc = jnp.full(
                        (16,), idx_v[pl.ds(k, 16)][0] & 127, jnp.int32
                    )
                    kk = jnp.full((16,), k & (bpw // 2 - 1), jnp.int32)
                    for g in range(DIM // 16):
                        rows = g * 16 + lanes
                        x = plsc.load_gather(rbuf.at[r], [rows, cc])
                        plsc.store_scatter(out_v, [kk, rows], x)

                    @pl.when(k < bpw - NBUF)
                    def _next(k=k, r=r):
                        fetch(k + NBUF, r)

            return carry

        half = bpw // 2
        lax.fori_loop(0, half // NBUF, body, 0)
        pltpu.sync_copy(out_v, out.at[pl.ds(base, half), :])
        lax.fori_loop(half // NBUF, bpw // NBUF, body, 0)
        pltpu.sync_copy(out_v, out.at[pl.ds(base + half, half), :])

    return gather_cols


def kernel(table, batch_idx):
    f = _build()
    return f(table.T, batch_idx.astype(jnp.int32))
